# baseline (speedup 1.0000x reference)
import functools

import jax
import jax.numpy as jnp
from jax import lax
from jax.experimental import pallas as pl
from jax.experimental.pallas import tpu as pltpu
from jax.experimental.pallas import tpu_sc as plsc

NC, NS = 2, 16
NW = NC * NS
LB = 32
NUM_T = 4
NUM_CLASSES = 10
TAU = 10.0
BATCH = 2048
BW = BATCH // 2
BCH = BATCH // LB
CHUNK = 16


def _enc_body(x_ref, o_ref):
    x = x_ref[...]
    t = lax.broadcasted_iota(jnp.int32, (1, NUM_T, 1), 1).astype(jnp.float32)
    thr = (t + 1.0) / (NUM_T + 1.0)
    xa = x[:, None, :BW]
    xb = x[:, None, BW:]
    ga = jnp.where(xa >= thr, jnp.int32(0x3F80), jnp.int32(0))
    gb = jnp.where(xb >= thr, jnp.int32(0x3F800000), jnp.int32(0))
    o_ref[...] = ga | gb


def _encode(xT):
    f = xT.shape[0]
    blk = 256
    return pl.pallas_call(
        _enc_body,
        grid=(f // blk,),
        in_specs=[pl.BlockSpec((blk, BATCH), lambda i: (i, 0))],
        out_specs=pl.BlockSpec((blk, NUM_T, BW), lambda i: (i, 0, 0)),
        out_shape=jax.ShapeDtypeStruct((f, NUM_T, BW), jnp.int32),
    )(xT)


def _coef_body(scale, l_ref, o_ref):
    om = 0.5 + 0.5 * jnp.sin(l_ref[...])
    w00, w01, w10, w11 = om[0:1], om[1:2], om[2:3], om[3:4]
    a = w00
    b = w10 - w00
    c = w01 - w00
    d = w00 - w01 - w10 + w11
    o_ref[...] = jnp.concatenate([a, b, c, d], axis=0) * scale


def _coefs(logitsT, scale):
    return pl.pallas_call(
        functools.partial(_coef_body, scale),
        out_shape=jax.ShapeDtypeStruct(logitsT.shape, jnp.float32),
    )(logitsT)


def _head_body(p_ref, o_ref):
    o_ref[...] = jnp.sum(p_ref[...], axis=0)


def _head_sum(parts):
    return pl.pallas_call(
        _head_body,
        out_shape=jax.ShapeDtypeStruct((NUM_CLASSES, 2, BATCH // 2),
                                       jnp.float32),
    )(parts)


def _prep(left, right, logitsT, w_pad, scale):
    w = left.shape[0]
    pad = w_pad - w
    zi = jnp.zeros((pad,), jnp.int32)
    lp = jnp.concatenate([left, zi])
    rp = jnp.concatenate([right, zi])
    cf = _coefs(logitsT, scale)
    cfp = jnp.concatenate([cf, jnp.zeros((4, pad), jnp.float32)], axis=1)
    return lp, rp, cfp


def _mesh():
    return plsc.VectorSubcoreMesh(core_axis_name="c", subcore_axis_name="s")


def _splat_bf16(s):
    v = jnp.full((16,), s, jnp.float32)
    u = plsc.bitcast(v, jnp.uint32)
    hb = (u + jnp.uint32(0x8000)) >> 16
    return plsc.bitcast(hb | (hb << 16), jnp.bfloat16)


def _chunk_coefs(cf, g):
    av = cf[0, pl.ds(g, CHUNK)]
    bv = cf[1, pl.ds(g, CHUNK)]
    cv = cf[2, pl.ds(g, CHUNK)]
    dv = cf[3, pl.ds(g, CHUNK)]
    return [(_splat_bf16(av[j]), _splat_bf16(bv[j]),
             _splat_bf16(cv[j]), _splat_bf16(dv[j])) for j in range(CHUNK)]


def _gate_layer(hin, left_p, right_p, coef_p, w_pad):
    gpt = w_pad // NW
    nch = gpt // CHUNK

    nbuf = 3
    assert nch % nbuf == 0

    @functools.partial(
        pl.kernel,
        out_type=jax.ShapeDtypeStruct((w_pad, BW), jnp.int32),
        mesh=_mesh(),
        compiler_params=pltpu.CompilerParams(needs_layout_passes=False),
        scratch_types=[
            pltpu.VMEM((gpt,), jnp.int32),
            pltpu.VMEM((gpt,), jnp.int32),
            pltpu.VMEM((4, gpt), jnp.float32),
            pltpu.VMEM((nbuf, CHUNK, BW), jnp.int32),
            pltpu.VMEM((nbuf, CHUNK, BW), jnp.int32),
            pltpu.VMEM((CHUNK, BW), jnp.int32),
            pltpu.SemaphoreType.DMA,
            pltpu.SemaphoreType.DMA,
            pltpu.SemaphoreType.DMA,
            pltpu.SemaphoreType.DMA,
            pltpu.SemaphoreType.DMA,
            pltpu.SemaphoreType.DMA,
        ],
    )
    def k(hin_h, left_h, right_h, coef_h, out_h,
          idxl, idxr, cf, lbs, rbs, obuf, *sems):
        sls, srs = sems[:nbuf], sems[nbuf:]
        wid = lax.axis_index("s") * NC + lax.axis_index("c")
        base = wid * gpt
        pltpu.sync_copy(left_h.at[pl.ds(base, gpt)], idxl)
        pltpu.sync_copy(right_h.at[pl.ds(base, gpt)], idxr)
        pltpu.sync_copy(coef_h.at[:, pl.ds(base, gpt)], cf)

        def gather(c, p):
            g = c * CHUNK
            pltpu.async_copy(hin_h.at[idxl.at[pl.ds(g, CHUNK)]],
                             lbs.at[p], sls[p])
            pltpu.async_copy(hin_h.at[idxr.at[pl.ds(g, CHUNK)]],
                             rbs.at[p], srs[p])

        def wait(p):
            pltpu.make_async_copy(hin_h.at[idxl.at[pl.ds(0, CHUNK)]],
                                  lbs.at[p], sls[p]).wait()
            pltpu.make_async_copy(hin_h.at[idxr.at[pl.ds(0, CHUNK)]],
                                  rbs.at[p], srs[p]).wait()

        def compute_store(c, p):
            g = c * CHUNK
            av = cf[0, pl.ds(g, CHUNK)]
            bv = cf[1, pl.ds(g, CHUNK)]
            cv = cf[2, pl.ds(g, CHUNK)]
            dv = cf[3, pl.ds(g, CHUNK)]
            lb = lbs.at[p]
            rb = rbs.at[p]

            for grp in range(0, CHUNK, 8):
                scg = [(_splat_bf16(av[j]), _splat_bf16(bv[j]),
                        _splat_bf16(cv[j]), _splat_bf16(dv[j]))
                       for j in range(grp, grp + 8)]

                def bstep(bi, _, grp=grp, scg=scg):
                    s = bi * 16
                    for jj, (a, b, cc, d) in enumerate(scg):
                        j = grp + jj
                        l = plsc.bitcast(lb[j, pl.ds(s, 16)], jnp.bfloat16)
                        r = plsc.bitcast(rb[j, pl.ds(s, 16)], jnp.bfloat16)
                        o = (a + b * l) + r * (cc + d * l)
                        obuf[j, pl.ds(s, 16)] = plsc.bitcast(o, jnp.int32)
                    return 0

                lax.fori_loop(0, BCH, bstep, 0, unroll=2)
            pltpu.sync_copy(obuf, out_h.at[pl.ds(base + g, CHUNK)])

        for p in range(nbuf - 1):
            gather(p, p)

        def body(it, _):
            c0 = it * nbuf
            for p in range(nbuf):
                c = c0 + p
                nxt = c + nbuf - 1

                @pl.when(nxt < nch)
                def _():
                    gather(nxt, (p - 1) % nbuf)

                wait(p)
                compute_store(c, p)
            return 0

        lax.fori_loop(0, nch // nbuf, body, 0)

    return k(hin, left_p, right_p, coef_p)


def _gate_head_layer(hin, left_p, right_p, coef_p, w_pad):
    gpt = w_pad // NW
    nch = gpt // CHUNK

    @functools.partial(
        pl.kernel,
        out_type=jax.ShapeDtypeStruct((NW, NUM_CLASSES, 2, BATCH // 2),
                                      jnp.float32),
        mesh=_mesh(),
        compiler_params=pltpu.CompilerParams(needs_layout_passes=False),
        scratch_types=[
            pltpu.VMEM((gpt,), jnp.int32),
            pltpu.VMEM((gpt,), jnp.int32),
            pltpu.VMEM((4, gpt), jnp.float32),
            pltpu.VMEM((CHUNK, BW), jnp.int32),
            pltpu.VMEM((CHUNK, BW), jnp.int32),
            pltpu.VMEM((CHUNK, BW), jnp.int32),
            pltpu.VMEM((CHUNK, BW), jnp.int32),
            pltpu.VMEM((NUM_CLASSES, 2, BATCH // 2), jnp.float32),
            pltpu.SemaphoreType.DMA,
            pltpu.SemaphoreType.DMA,
            pltpu.SemaphoreType.DMA,
            pltpu.SemaphoreType.DMA,
        ],
    )
    def k(hin_h, left_h, right_h, coef_h, out_h,
          idxl, idxr, cf, lb0, rb0, lb1, rb1, acc,
          sl0, sr0, sl1, sr1):
        wid = lax.axis_index("s") * NC + lax.axis_index("c")
        base = wid * gpt
        pltpu.sync_copy(left_h.at[pl.ds(base, gpt)], idxl)
        pltpu.sync_copy(right_h.at[pl.ds(base, gpt)], idxr)
        pltpu.sync_copy(coef_h.at[:, pl.ds(base, gpt)], cf)

        def zrow(i, _):
            def zcol(bi, _):
                z = jnp.zeros((16,), jnp.float32)
                acc[i, 0, pl.ds(bi * 16, 16)] = z
                acc[i, 1, pl.ds(bi * 16, 16)] = z
                return 0
            lax.fori_loop(0, BCH, zcol, 0, unroll=4)
            return 0

        lax.fori_loop(0, NUM_CLASSES, zrow, 0)

        def gather(c, lb, rb, sl, sr):
            g = c * CHUNK
            return (pltpu.async_copy(hin_h.at[idxl.at[pl.ds(g, CHUNK)]],
                                     lb, sl),
                    pltpu.async_copy(hin_h.at[idxr.at[pl.ds(g, CHUNK)]],
                                     rb, sr))

        def compute_acc(c, lb, rb):
            g = c * CHUNK
            av = cf[0, pl.ds(g, CHUNK)]
            bv = cf[1, pl.ds(g, CHUNK)]
            cv = cf[2, pl.ds(g, CHUNK)]
            dv = cf[3, pl.ds(g, CHUNK)]
            cls = lax.min((base + g) // 1600, NUM_CLASSES - 1)

            for grp in range(0, CHUNK, 8):
                scg = [(_splat_bf16(av[j]), _splat_bf16(bv[j]),
                        _splat_bf16(cv[j]), _splat_bf16(dv[j]))
                       for j in range(grp, grp + 8)]

                def bstep(bi, _, grp=grp, scg=scg):
                    s = bi * 16
                    tot0 = jnp.zeros((16,), jnp.float32)
                    tot1 = jnp.zeros((16,), jnp.float32)
                    for jj, (a, b, cc, d) in enumerate(scg):
                        j = grp + jj
                        l = plsc.bitcast(lb[j, pl.ds(s, 16)], jnp.bfloat16)
                        r = plsc.bitcast(rb[j, pl.ds(s, 16)], jnp.bfloat16)
                        o = (a + b * l) + r * (cc + d * l)
                        u = plsc.bitcast(o, jnp.uint32)
                        tot0 = tot0 + plsc.bitcast(u << 16, jnp.float32)
                        tot1 = tot1 + plsc.bitcast(
                            u & jnp.uint32(0xFFFF0000), jnp.float32)
                    acc[cls, 0, pl.ds(s, 16)] = (
                        acc[cls, 0, pl.ds(s, 16)] + tot0)
                    acc[cls, 1, pl.ds(s, 16)] = (
                        acc[cls, 1, pl.ds(s, 16)] + tot1)
                    return 0

                lax.fori_loop(0, BCH, bstep, 0, unroll=2)

        gather(0, lb0, rb0, sl0, sr0)

        def body(ci2, _):
            c0 = ci2 * 2
            w1 = gather(c0 + 1, lb1, rb1, sl1, sr1)
            pltpu.make_async_copy(hin_h.at[idxl.at[pl.ds(0, CHUNK)]],
                                  lb0, sl0).wait()
            pltpu.make_async_copy(hin_h.at[idxr.at[pl.ds(0, CHUNK)]],
                                  rb0, sr0).wait()
            compute_acc(c0, lb0, rb0)

            @pl.when(ci2 * 2 + 2 < nch)
            def _():
                gather(c0 + 2, lb0, rb0, sl0, sr0)

            w1[0].wait()
            w1[1].wait()
            compute_acc(c0 + 1, lb1, rb1)
            return 0

        lax.fori_loop(0, nch // 2, body, 0)
        pltpu.sync_copy(acc, out_h.at[wid])

    return k(hin, left_p, right_p, coef_p)


WP01 = 24576
WP2 = 16384


def kernel(x, left0, right0, logits0, left1, right1, logits1,
           left2, right2, logits2):
    xT = x.reshape(BATCH, -1).T
    h0 = _encode(xT).reshape(-1, BW)

    l0, r0, c0 = _prep(left0, right0, logits0.T, WP01, 1.0)
    l1, r1, c1 = _prep(left1, right1, logits1.T, WP01, 1.0)
    l2, r2, c2 = _prep(left2, right2, logits2.T, WP2, 1.0 / TAU)

    h1 = _gate_layer(h0, l0, r0, c0, WP01)
    h2 = _gate_layer(h1, l1, r1, c1, WP01)
    parts = _gate_head_layer(h2, l2, r2, c2, WP2)
    s = _head_sum(parts)
    return s.reshape(NUM_CLASSES, BATCH).T


# speedup vs baseline: 1.0878x; 1.0878x over previous
import functools

import jax
import jax.numpy as jnp
from jax import lax
from jax.experimental import pallas as pl
from jax.experimental.pallas import tpu as pltpu
from jax.experimental.pallas import tpu_sc as plsc

NC, NS = 2, 16
NW = NC * NS
LB = 32
NUM_T = 4
NUM_CLASSES = 10
TAU = 10.0
BATCH = 2048
BW = BATCH // 2
BCH = BATCH // LB
CHUNK = 16


def _enc_body(x_ref, o_ref):
    x = x_ref[...]
    t = lax.broadcasted_iota(jnp.int32, (1, NUM_T, 1), 1).astype(jnp.float32)
    thr = (t + 1.0) / (NUM_T + 1.0)
    xa = x[:, None, :BW]
    xb = x[:, None, BW:]
    ga = jnp.where(xa >= thr, jnp.int32(0x3F80), jnp.int32(0))
    gb = jnp.where(xb >= thr, jnp.int32(0x3F800000), jnp.int32(0))
    o_ref[...] = ga | gb


def _encode(xT):
    f = xT.shape[0]
    blk = 256
    return pl.pallas_call(
        _enc_body,
        grid=(f // blk,),
        in_specs=[pl.BlockSpec((blk, BATCH), lambda i: (i, 0))],
        out_specs=pl.BlockSpec((blk, NUM_T, BW), lambda i: (i, 0, 0)),
        out_shape=jax.ShapeDtypeStruct((f, NUM_T, BW), jnp.int32),
    )(xT)


def _coef_body(scale, l_ref, o_ref):
    om = 0.5 + 0.5 * jnp.sin(l_ref[...])
    w00, w01, w10, w11 = om[0:1], om[1:2], om[2:3], om[3:4]
    a = w00
    b = w10 - w00
    c = w01 - w00
    d = w00 - w01 - w10 + w11
    o_ref[...] = jnp.concatenate([a, b, c, d], axis=0) * scale


def _coefs(logitsT, scale):
    return pl.pallas_call(
        functools.partial(_coef_body, scale),
        out_shape=jax.ShapeDtypeStruct(logitsT.shape, jnp.float32),
    )(logitsT)


def _head_body(p_ref, o_ref):
    o_ref[...] = jnp.sum(p_ref[...], axis=0)


def _head_sum(parts):
    return pl.pallas_call(
        _head_body,
        out_shape=jax.ShapeDtypeStruct((NUM_CLASSES, 2, BATCH // 2),
                                       jnp.float32),
    )(parts)


def _prep(left, right, logitsT, w_pad, scale):
    w = left.shape[0]
    pad = w_pad - w
    zi = jnp.zeros((pad,), jnp.int32)
    lp = jnp.concatenate([left, zi])
    rp = jnp.concatenate([right, zi])
    cf = _coefs(logitsT, scale)
    cfp = jnp.concatenate([cf, jnp.zeros((4, pad), jnp.float32)], axis=1)
    return lp, rp, cfp


def _mesh():
    return plsc.VectorSubcoreMesh(core_axis_name="c", subcore_axis_name="s")


def _splat_bf16(s):
    v = jnp.full((16,), s, jnp.float32)
    u = plsc.bitcast(v, jnp.uint32)
    hb = (u + jnp.uint32(0x8000)) >> 16
    return plsc.bitcast(hb | (hb << 16), jnp.bfloat16)


def _chunk_coefs(cf, g):
    av = cf[0, pl.ds(g, CHUNK)]
    bv = cf[1, pl.ds(g, CHUNK)]
    cv = cf[2, pl.ds(g, CHUNK)]
    dv = cf[3, pl.ds(g, CHUNK)]
    return [(_splat_bf16(av[j]), _splat_bf16(bv[j]),
             _splat_bf16(cv[j]), _splat_bf16(dv[j])) for j in range(CHUNK)]


def _gate_layer(hin, left_p, right_p, coef_p, w_pad):
    gpt = w_pad // NW
    nch = gpt // CHUNK

    nbuf = 3
    assert nch % nbuf == 0

    @functools.partial(
        pl.kernel,
        out_type=jax.ShapeDtypeStruct((w_pad, BW), jnp.int32),
        mesh=_mesh(),
        compiler_params=pltpu.CompilerParams(needs_layout_passes=False),
        scratch_types=[
            pltpu.VMEM((gpt,), jnp.int32),
            pltpu.VMEM((gpt,), jnp.int32),
            pltpu.VMEM((4, gpt), jnp.float32),
            pltpu.VMEM((nbuf, CHUNK, BW), jnp.int32),
            pltpu.VMEM((nbuf, CHUNK, BW), jnp.int32),
            pltpu.VMEM((CHUNK, BW), jnp.int32),
            pltpu.SemaphoreType.DMA,
            pltpu.SemaphoreType.DMA,
            pltpu.SemaphoreType.DMA,
            pltpu.SemaphoreType.DMA,
            pltpu.SemaphoreType.DMA,
            pltpu.SemaphoreType.DMA,
        ],
    )
    def k(hin_h, left_h, right_h, coef_h, out_h,
          idxl, idxr, cf, lbs, rbs, obuf, *sems):
        sls, srs = sems[:nbuf], sems[nbuf:]
        wid = lax.axis_index("s") * NC + lax.axis_index("c")
        base = wid * gpt
        pltpu.sync_copy(left_h.at[pl.ds(base, gpt)], idxl)
        pltpu.sync_copy(right_h.at[pl.ds(base, gpt)], idxr)
        pltpu.sync_copy(coef_h.at[:, pl.ds(base, gpt)], cf)

        def gather(c, p):
            g = c * CHUNK
            pltpu.async_copy(hin_h.at[idxl.at[pl.ds(g, CHUNK)]],
                             lbs.at[p], sls[p])
            pltpu.async_copy(hin_h.at[idxr.at[pl.ds(g, CHUNK)]],
                             rbs.at[p], srs[p])

        def wait(p):
            pltpu.make_async_copy(hin_h.at[idxl.at[pl.ds(0, CHUNK)]],
                                  lbs.at[p], sls[p]).wait()
            pltpu.make_async_copy(hin_h.at[idxr.at[pl.ds(0, CHUNK)]],
                                  rbs.at[p], srs[p]).wait()

        def compute_store(c, p):
            g = c * CHUNK
            av = cf[0, pl.ds(g, CHUNK)]
            bv = cf[1, pl.ds(g, CHUNK)]
            cv = cf[2, pl.ds(g, CHUNK)]
            dv = cf[3, pl.ds(g, CHUNK)]
            lb = lbs.at[p]
            rb = rbs.at[p]

            for grp in range(0, CHUNK, 8):
                scg = [(_splat_bf16(av[j]), _splat_bf16(bv[j]),
                        _splat_bf16(cv[j]), _splat_bf16(dv[j]))
                       for j in range(grp, grp + 8)]

                def bstep(bi, _, grp=grp, scg=scg):
                    s = bi * 16
                    for jj, (a, b, cc, d) in enumerate(scg):
                        j = grp + jj
                        l = plsc.bitcast(lb[j, pl.ds(s, 16)], jnp.bfloat16)
                        r = plsc.bitcast(rb[j, pl.ds(s, 16)], jnp.bfloat16)
                        o = (a + b * l) + r * (cc + d * l)
                        obuf[j, pl.ds(s, 16)] = plsc.bitcast(o, jnp.int32)
                    return 0

                lax.fori_loop(0, BCH, bstep, 0, unroll=1)
            pltpu.sync_copy(obuf, out_h.at[pl.ds(base + g, CHUNK)])

        for p in range(nbuf - 1):
            gather(p, p)

        def body(it, _):
            c0 = it * nbuf
            for p in range(nbuf):
                c = c0 + p
                nxt = c + nbuf - 1

                @pl.when(nxt < nch)
                def _():
                    gather(nxt, (p - 1) % nbuf)

                wait(p)
                compute_store(c, p)
            return 0

        lax.fori_loop(0, nch // nbuf, body, 0)

    return k(hin, left_p, right_p, coef_p)


def _gate_head_layer(hin, left_p, right_p, coef_p, w_pad):
    gpt = w_pad // NW
    nch = gpt // CHUNK

    @functools.partial(
        pl.kernel,
        out_type=jax.ShapeDtypeStruct((NW, NUM_CLASSES, 2, BATCH // 2),
                                      jnp.float32),
        mesh=_mesh(),
        compiler_params=pltpu.CompilerParams(needs_layout_passes=False),
        scratch_types=[
            pltpu.VMEM((gpt,), jnp.int32),
            pltpu.VMEM((gpt,), jnp.int32),
            pltpu.VMEM((4, gpt), jnp.float32),
            pltpu.VMEM((CHUNK, BW), jnp.int32),
            pltpu.VMEM((CHUNK, BW), jnp.int32),
            pltpu.VMEM((CHUNK, BW), jnp.int32),
            pltpu.VMEM((CHUNK, BW), jnp.int32),
            pltpu.VMEM((NUM_CLASSES, 2, BATCH // 2), jnp.float32),
            pltpu.SemaphoreType.DMA,
            pltpu.SemaphoreType.DMA,
            pltpu.SemaphoreType.DMA,
            pltpu.SemaphoreType.DMA,
        ],
    )
    def k(hin_h, left_h, right_h, coef_h, out_h,
          idxl, idxr, cf, lb0, rb0, lb1, rb1, acc,
          sl0, sr0, sl1, sr1):
        wid = lax.axis_index("s") * NC + lax.axis_index("c")
        base = wid * gpt
        pltpu.sync_copy(left_h.at[pl.ds(base, gpt)], idxl)
        pltpu.sync_copy(right_h.at[pl.ds(base, gpt)], idxr)
        pltpu.sync_copy(coef_h.at[:, pl.ds(base, gpt)], cf)

        def zrow(i, _):
            def zcol(bi, _):
                z = jnp.zeros((16,), jnp.float32)
                acc[i, 0, pl.ds(bi * 16, 16)] = z
                acc[i, 1, pl.ds(bi * 16, 16)] = z
                return 0
            lax.fori_loop(0, BCH, zcol, 0, unroll=4)
            return 0

        lax.fori_loop(0, NUM_CLASSES, zrow, 0)

        def gather(c, lb, rb, sl, sr):
            g = c * CHUNK
            return (pltpu.async_copy(hin_h.at[idxl.at[pl.ds(g, CHUNK)]],
                                     lb, sl),
                    pltpu.async_copy(hin_h.at[idxr.at[pl.ds(g, CHUNK)]],
                                     rb, sr))

        def compute_acc(c, lb, rb):
            g = c * CHUNK
            av = cf[0, pl.ds(g, CHUNK)]
            bv = cf[1, pl.ds(g, CHUNK)]
            cv = cf[2, pl.ds(g, CHUNK)]
            dv = cf[3, pl.ds(g, CHUNK)]
            cls = lax.min((base + g) // 1600, NUM_CLASSES - 1)

            for grp in range(0, CHUNK, 8):
                scg = [(_splat_bf16(av[j]), _splat_bf16(bv[j]),
                        _splat_bf16(cv[j]), _splat_bf16(dv[j]))
                       for j in range(grp, grp + 8)]

                def bstep(bi, _, grp=grp, scg=scg):
                    s = bi * 16
                    tot0 = jnp.zeros((16,), jnp.float32)
                    tot1 = jnp.zeros((16,), jnp.float32)
                    for jj, (a, b, cc, d) in enumerate(scg):
                        j = grp + jj
                        l = plsc.bitcast(lb[j, pl.ds(s, 16)], jnp.bfloat16)
                        r = plsc.bitcast(rb[j, pl.ds(s, 16)], jnp.bfloat16)
                        o = (a + b * l) + r * (cc + d * l)
                        u = plsc.bitcast(o, jnp.uint32)
                        tot0 = tot0 + plsc.bitcast(u << 16, jnp.float32)
                        tot1 = tot1 + plsc.bitcast(
                            u & jnp.uint32(0xFFFF0000), jnp.float32)
                    acc[cls, 0, pl.ds(s, 16)] = (
                        acc[cls, 0, pl.ds(s, 16)] + tot0)
                    acc[cls, 1, pl.ds(s, 16)] = (
                        acc[cls, 1, pl.ds(s, 16)] + tot1)
                    return 0

                lax.fori_loop(0, BCH, bstep, 0, unroll=1)

        gather(0, lb0, rb0, sl0, sr0)

        def body(ci2, _):
            c0 = ci2 * 2
            w1 = gather(c0 + 1, lb1, rb1, sl1, sr1)
            pltpu.make_async_copy(hin_h.at[idxl.at[pl.ds(0, CHUNK)]],
                                  lb0, sl0).wait()
            pltpu.make_async_copy(hin_h.at[idxr.at[pl.ds(0, CHUNK)]],
                                  rb0, sr0).wait()
            compute_acc(c0, lb0, rb0)

            @pl.when(ci2 * 2 + 2 < nch)
            def _():
                gather(c0 + 2, lb0, rb0, sl0, sr0)

            w1[0].wait()
            w1[1].wait()
            compute_acc(c0 + 1, lb1, rb1)
            return 0

        lax.fori_loop(0, nch // 2, body, 0)
        pltpu.sync_copy(acc, out_h.at[wid])

    return k(hin, left_p, right_p, coef_p)


WP01 = 24576
WP2 = 16384


def kernel(x, left0, right0, logits0, left1, right1, logits1,
           left2, right2, logits2):
    xT = x.reshape(BATCH, -1).T
    h0 = _encode(xT).reshape(-1, BW)

    l0, r0, c0 = _prep(left0, right0, logits0.T, WP01, 1.0)
    l1, r1, c1 = _prep(left1, right1, logits1.T, WP01, 1.0)
    l2, r2, c2 = _prep(left2, right2, logits2.T, WP2, 1.0 / TAU)

    h1 = _gate_layer(h0, l0, r0, c0, WP01)
    h2 = _gate_layer(h1, l1, r1, c1, WP01)
    parts = _gate_head_layer(h2, l2, r2, c2, WP2)
    s = _head_sum(parts)
    return s.reshape(NUM_CLASSES, BATCH).T
